# baseline (speedup 1.0000x reference)
import functools

import jax
import jax.numpy as jnp
from jax import lax
from jax.experimental import pallas as pl
from jax.experimental.pallas import tpu as pltpu
from jax.experimental.pallas import tpu_sc as plsc

N_NODES = 10000
N_EDGES = 320000
D_FEAT = 128
D_HALF = 64
N_CLASSES = 40
C_PAD = 48
C_STRIDE = 56
D_STRIDE = 72

NC = 2
NS = 16
NW = NC * NS
L = 16
K = 112
NBUF = 4
NBI = 8
N_PAD = 10240
ROWS_PER_TILE = N_PAD // NS
ZSIZES = (112, 112, 112, 112, 112, 80)

NCH1 = -(-N_EDGES // (NS * K))
E1_PAD = NS * K * NCH1
NCH2 = -(-N_EDGES // (NW * K))
E2_PAD = NW * K * NCH2


def _make_edge_aggregate(d, nch, col_split, nsv):
    mesh = plsc.VectorSubcoreMesh(core_axis_name="c", subcore_axis_name="s")

    @functools.partial(
        pl.kernel,
        out_type=jax.ShapeDtypeStruct((NC, N_PAD, d), jnp.float32),
        mesh=mesh,
        scratch_types=[
            pltpu.VMEM((NBI, K), jnp.int32),
            pltpu.VMEM((NBI, K), jnp.int32),
            pltpu.VMEM((NBI, K), jnp.float32),
            pltpu.VMEM((NBUF, K, d), jnp.float32),
            pltpu.VMEM_SHARED((N_PAD, d), jnp.float32),
            pltpu.VMEM_SHARED((N_PAD, d), jnp.float32),
            pltpu.SemaphoreType.DMA,
            pltpu.SemaphoreType.DMA,
            pltpu.SemaphoreType.DMA,
        ],
        compiler_params=pltpu.CompilerParams(
            needs_layout_passes=False,
            use_tc_tiling_on_sc=False,
        ),
    )
    def body(feat_hbm, src_hbm, dst_hbm, w_hbm, out_hbm,
             srcv, dstv, wv, rows, table, acc, gsem, ssem, isem):
        cid = lax.axis_index("c")
        sid = lax.axis_index("s")
        widx = sid if col_split else sid * NC + cid
        tbase = sid * ROWS_PER_TILE
        zero16 = jnp.zeros((L,), jnp.float32)

        def fetch_idx(i, slot):
            pltpu.async_copy(src_hbm.at[widx, i], srcv.at[slot], isem)
            pltpu.async_copy(dst_hbm.at[widx, i], dstv.at[slot], isem)
            pltpu.async_copy(w_hbm.at[widx, i], wv.at[slot], isem)

        def wait_idx(i, slot):
            pltpu.make_async_copy(src_hbm.at[widx, i], srcv.at[slot],
                                  isem).wait()
            pltpu.make_async_copy(dst_hbm.at[widx, i], dstv.at[slot],
                                  isem).wait()
            pltpu.make_async_copy(w_hbm.at[widx, i], wv.at[slot],
                                  isem).wait()

        fetch_idx(0, 0)
        fetch_idx(1, 1)
        fetch_idx(2, 2)

        if col_split:
            pltpu.sync_copy(feat_hbm.at[cid, pl.ds(tbase, ROWS_PER_TILE)],
                            table.at[pl.ds(tbase, ROWS_PER_TILE)])
        else:
            pltpu.sync_copy(feat_hbm.at[pl.ds(tbase, ROWS_PER_TILE)],
                            table.at[pl.ds(tbase, ROWS_PER_TILE)])

        def zbody(j, _):
            for t in range(d // L):
                rows[0, j, pl.ds(t * L, L)] = zero16
            return 0
        lax.fori_loop(0, K, zbody, 0)
        off = 0
        for z in ZSIZES:
            pltpu.sync_copy(rows.at[0, pl.ds(0, z)],
                            acc.at[pl.ds(tbase + off, z)])
            off += z
        plsc.subcore_barrier()

        def chunk(t, _):
            i = t - 1

            @pl.when(jnp.logical_and(i >= 3, i - 3 < nch))
            def _():
                pltpu.make_async_copy(rows.at[lax.rem(i - 3, NBUF)],
                                      acc.at[dstv.at[lax.rem(i - 3, NBI)]],
                                      ssem).wait()

            @pl.when(i + 1 < nch)
            def _():
                g = i + 1
                gslot = lax.rem(g, NBI)
                wait_idx(g, gslot)
                pltpu.async_copy(table.at[srcv.at[gslot]],
                                 rows.at[lax.rem(g, NBUF)], gsem)

            @pl.when(i + 4 < nch)
            def _():
                fetch_idx(i + 4, lax.rem(i + 4, NBI))

            @pl.when(jnp.logical_and(i >= 1, i - 1 < nch))
            def _():
                j = i - 1
                pslot = lax.rem(j, NBI)
                pbuf = lax.rem(j, NBUF)
                pltpu.make_async_copy(table.at[srcv.at[pslot]],
                                      rows.at[pbuf], gsem).wait()

                def gbody(gg, _):
                    base = gg * L
                    for jj in range(L):
                        e = base + jj
                        ws = plsc.load_gather(
                            wv, [jnp.full((L,), pslot, jnp.int32),
                                 jnp.full((L,), e, jnp.int32)])
                        for tt in range(nsv):
                            rows[pbuf, e, pl.ds(tt * L, L)] = (
                                rows[pbuf, e, pl.ds(tt * L, L)] * ws)
                    return 0
                lax.fori_loop(0, K // L, gbody, 0)

                pltpu.async_copy(rows.at[pbuf], acc.at[dstv.at[pslot]], ssem,
                                 add=True)
            return 0
        lax.fori_loop(0, nch + 2, chunk, 0)
        for jj in (nch - 2, nch - 1):
            pltpu.make_async_copy(rows.at[jj % NBUF],
                                  acc.at[dstv.at[jj % NBI]], ssem).wait()

        plsc.subcore_barrier()
        off = 0
        for z in ZSIZES:
            pltpu.sync_copy(acc.at[pl.ds(tbase + off, z)],
                            out_hbm.at[cid, pl.ds(tbase + off, z)])
            off += z

    return body


_agg1 = _make_edge_aggregate(D_STRIDE, NCH1, col_split=True, nsv=D_HALF // L)
_agg2 = _make_edge_aggregate(C_STRIDE, NCH2, col_split=False, nsv=C_PAD // L)


def _mm1(xp, w1s):
    def body(x_ref, w_ref, o_ref):
        o_ref[0] = jnp.dot(x_ref[...], w_ref[0],
                           preferred_element_type=jnp.float32)
    return pl.pallas_call(
        body,
        grid=(2, 8),
        in_specs=[pl.BlockSpec((1280, D_FEAT), lambda c, i: (i, 0)),
                  pl.BlockSpec((1, D_FEAT, D_STRIDE), lambda c, i: (c, 0, 0))],
        out_specs=pl.BlockSpec((1, 1280, D_STRIDE), lambda c, i: (c, i, 0)),
        out_shape=jax.ShapeDtypeStruct((NC, N_PAD, D_STRIDE), jnp.float32),
    )(xp, w1s)


def _relu_mm(p, b1e, w2e):
    def body(p_ref, b_ref, w_ref, o_ref):
        h = jnp.concatenate([p_ref[0], p_ref[1]], axis=-1)
        h = jnp.maximum(h + b_ref[...], 0.0)
        o_ref[...] = jnp.dot(h, w_ref[...], preferred_element_type=jnp.float32)
    return pl.pallas_call(
        body,
        grid=(8,),
        in_specs=[pl.BlockSpec((2, 1280, D_STRIDE), lambda i: (0, i, 0)),
                  pl.BlockSpec((1, 2 * D_STRIDE), lambda i: (0, 0)),
                  pl.BlockSpec((2 * D_STRIDE, C_STRIDE), lambda i: (0, 0))],
        out_specs=pl.BlockSpec((1280, C_STRIDE), lambda i: (i, 0)),
        out_shape=jax.ShapeDtypeStruct((N_PAD, C_STRIDE), jnp.float32),
    )(p, b1e.reshape(1, 2 * D_STRIDE), w2e)


def _lsm(q, b2p):
    def body(q_ref, b_ref, o_ref):
        z = q_ref[0] + q_ref[1] + b_ref[...]
        m = jnp.max(z, axis=1, keepdims=True)
        e = jnp.exp(z - m)
        o_ref[...] = z - m - jnp.log(jnp.sum(e, axis=1, keepdims=True))
    return pl.pallas_call(
        body,
        grid=(8,),
        in_specs=[pl.BlockSpec((2, 1280, C_STRIDE), lambda i: (0, i, 0)),
                  pl.BlockSpec((1, C_STRIDE), lambda i: (0, 0))],
        out_specs=pl.BlockSpec((1280, C_STRIDE), lambda i: (i, 0)),
        out_shape=jax.ShapeDtypeStruct((N_PAD, C_STRIDE), jnp.float32),
    )(q, b2p.reshape(1, C_STRIDE))


def kernel(x, edge_index, edge_weight, W1, b1, W2, b2):
    src = edge_index[0].astype(jnp.int32)
    dst = edge_index[1].astype(jnp.int32)
    w = edge_weight.astype(jnp.float32)

    src1 = jnp.pad(src, (0, E1_PAD - N_EDGES)).reshape(NS, NCH1, K)
    dst1 = jnp.pad(dst, (0, E1_PAD - N_EDGES)).reshape(NS, NCH1, K)
    w1e = jnp.pad(w, (0, E1_PAD - N_EDGES)).reshape(NS, NCH1, K)
    src2 = jnp.pad(src, (0, E2_PAD - N_EDGES)).reshape(NW, NCH2, K)
    dst2 = jnp.pad(dst, (0, E2_PAD - N_EDGES)).reshape(NW, NCH2, K)
    w2e = jnp.pad(w, (0, E2_PAD - N_EDGES)).reshape(NW, NCH2, K)

    xp = jnp.pad(x, ((0, N_PAD - N_NODES), (0, 0)))
    w1s = W1.reshape(D_FEAT, NC, D_HALF).transpose(1, 0, 2)
    w1s = jnp.pad(w1s, ((0, 0), (0, 0), (0, D_STRIDE - D_HALF)))
    t1 = _mm1(xp, w1s)
    p = _agg1(t1, src1, dst1, w1e)
    w2c = jnp.pad(W2, ((0, 0), (0, C_STRIDE - N_CLASSES)))
    w2x = jnp.zeros((2 * D_STRIDE, C_STRIDE), jnp.float32)
    w2x = w2x.at[0:D_HALF].set(w2c[0:D_HALF])
    w2x = w2x.at[D_STRIDE:D_STRIDE + D_HALF].set(w2c[D_HALF:D_FEAT])
    b1e = jnp.zeros((2 * D_STRIDE,), jnp.float32)
    b1e = b1e.at[0:D_HALF].set(b1[0:D_HALF])
    b1e = b1e.at[D_STRIDE:D_STRIDE + D_HALF].set(b1[D_HALF:D_FEAT])
    t2 = _relu_mm(p, b1e, w2x)
    q = _agg2(t2, src2, dst2, w2e)
    b2p = jnp.pad(b2, (0, C_STRIDE - N_CLASSES), constant_values=-1e30)
    out = _lsm(q, b2p)
    return out[:N_NODES, :N_CLASSES]


# speedup vs baseline: 1.0351x; 1.0351x over previous
import functools

import jax
import jax.numpy as jnp
from jax import lax
from jax.experimental import pallas as pl
from jax.experimental.pallas import tpu as pltpu
from jax.experimental.pallas import tpu_sc as plsc

N_NODES = 10000
N_EDGES = 320000
D_FEAT = 128
D_HALF = 64
N_CLASSES = 40
C_PAD = 48
C_STRIDE = 48
D_STRIDE = 64

NC = 2
NS = 16
NW = NC * NS
L = 16
K = 160
NBUF = 4
NBI = 8
N_PAD = 10240
ROWS_PER_TILE = N_PAD // NS
ZSIZES = (160, 160, 160, 160)

NCH1 = -(-N_EDGES // (NS * K))
E1_PAD = NS * K * NCH1
NCH2 = -(-N_EDGES // (NW * K))
E2_PAD = NW * K * NCH2


def _make_edge_aggregate(d, nch, col_split, nsv):
    mesh = plsc.VectorSubcoreMesh(core_axis_name="c", subcore_axis_name="s")

    @functools.partial(
        pl.kernel,
        out_type=jax.ShapeDtypeStruct((NC, N_PAD, d), jnp.float32),
        mesh=mesh,
        scratch_types=[
            pltpu.VMEM((NBI, K), jnp.int32),
            pltpu.VMEM((NBI, K), jnp.int32),
            pltpu.VMEM((NBI, K), jnp.float32),
            pltpu.VMEM((NBUF, K, d), jnp.float32),
            pltpu.VMEM_SHARED((N_PAD, d), jnp.float32),
            pltpu.VMEM_SHARED((N_PAD, d), jnp.float32),
            pltpu.SemaphoreType.DMA,
            pltpu.SemaphoreType.DMA,
            pltpu.SemaphoreType.DMA,
        ],
        compiler_params=pltpu.CompilerParams(
            needs_layout_passes=False,
            use_tc_tiling_on_sc=False,
        ),
    )
    def body(feat_hbm, src_hbm, dst_hbm, w_hbm, out_hbm,
             srcv, dstv, wv, rows, table, acc, gsem, ssem, isem):
        cid = lax.axis_index("c")
        sid = lax.axis_index("s")
        widx = sid if col_split else sid * NC + cid
        tbase = sid * ROWS_PER_TILE
        zero16 = jnp.zeros((L,), jnp.float32)

        def fetch_idx(i, slot):
            pltpu.async_copy(src_hbm.at[widx, i], srcv.at[slot], isem)
            pltpu.async_copy(dst_hbm.at[widx, i], dstv.at[slot], isem)
            pltpu.async_copy(w_hbm.at[widx, i], wv.at[slot], isem)

        def wait_idx(i, slot):
            pltpu.make_async_copy(src_hbm.at[widx, i], srcv.at[slot],
                                  isem).wait()
            pltpu.make_async_copy(dst_hbm.at[widx, i], dstv.at[slot],
                                  isem).wait()
            pltpu.make_async_copy(w_hbm.at[widx, i], wv.at[slot],
                                  isem).wait()

        fetch_idx(0, 0)
        fetch_idx(1, 1)
        fetch_idx(2, 2)

        if col_split:
            pltpu.sync_copy(feat_hbm.at[cid, pl.ds(tbase, ROWS_PER_TILE)],
                            table.at[pl.ds(tbase, ROWS_PER_TILE)])
        else:
            pltpu.sync_copy(feat_hbm.at[pl.ds(tbase, ROWS_PER_TILE)],
                            table.at[pl.ds(tbase, ROWS_PER_TILE)])

        def zbody(j, _):
            for t in range(d // L):
                rows[0, j, pl.ds(t * L, L)] = zero16
            return 0
        lax.fori_loop(0, K, zbody, 0)
        off = 0
        for z in ZSIZES:
            pltpu.sync_copy(rows.at[0, pl.ds(0, z)],
                            acc.at[pl.ds(tbase + off, z)])
            off += z
        plsc.subcore_barrier()

        def chunk(t, _):
            i = t - 1

            @pl.when(jnp.logical_and(i >= 3, i - 3 < nch))
            def _():
                pltpu.make_async_copy(rows.at[lax.rem(i - 3, NBUF)],
                                      acc.at[dstv.at[lax.rem(i - 3, NBI)]],
                                      ssem).wait()

            @pl.when(i + 1 < nch)
            def _():
                g = i + 1
                gslot = lax.rem(g, NBI)
                wait_idx(g, gslot)
                pltpu.async_copy(table.at[srcv.at[gslot]],
                                 rows.at[lax.rem(g, NBUF)], gsem)

            @pl.when(i + 4 < nch)
            def _():
                fetch_idx(i + 4, lax.rem(i + 4, NBI))

            @pl.when(jnp.logical_and(i >= 1, i - 1 < nch))
            def _():
                j = i - 1
                pslot = lax.rem(j, NBI)
                pbuf = lax.rem(j, NBUF)
                pltpu.make_async_copy(table.at[srcv.at[pslot]],
                                      rows.at[pbuf], gsem).wait()

                def gbody(gg, _):
                    base = gg * L
                    for jj in range(L):
                        e = base + jj
                        ws = plsc.load_gather(
                            wv, [jnp.full((L,), pslot, jnp.int32),
                                 jnp.full((L,), e, jnp.int32)])
                        for tt in range(nsv):
                            rows[pbuf, e, pl.ds(tt * L, L)] = (
                                rows[pbuf, e, pl.ds(tt * L, L)] * ws)
                    return 0
                lax.fori_loop(0, K // L, gbody, 0)

                pltpu.async_copy(rows.at[pbuf], acc.at[dstv.at[pslot]], ssem,
                                 add=True)
            return 0
        lax.fori_loop(0, nch + 2, chunk, 0)
        for jj in (nch - 2, nch - 1):
            pltpu.make_async_copy(rows.at[jj % NBUF],
                                  acc.at[dstv.at[jj % NBI]], ssem).wait()

        plsc.subcore_barrier()
        off = 0
        for z in ZSIZES:
            pltpu.sync_copy(acc.at[pl.ds(tbase + off, z)],
                            out_hbm.at[cid, pl.ds(tbase + off, z)])
            off += z

    return body


_agg1 = _make_edge_aggregate(D_STRIDE, NCH1, col_split=True, nsv=D_HALF // L)
_agg2 = _make_edge_aggregate(C_STRIDE, NCH2, col_split=False, nsv=C_PAD // L)


def _mm1(xp, w1s):
    def body(x_ref, w_ref, o_ref):
        o_ref[0] = jnp.dot(x_ref[...], w_ref[0],
                           preferred_element_type=jnp.float32)
    return pl.pallas_call(
        body,
        grid=(2, 8),
        in_specs=[pl.BlockSpec((1280, D_FEAT), lambda c, i: (i, 0)),
                  pl.BlockSpec((1, D_FEAT, D_STRIDE), lambda c, i: (c, 0, 0))],
        out_specs=pl.BlockSpec((1, 1280, D_STRIDE), lambda c, i: (c, i, 0)),
        out_shape=jax.ShapeDtypeStruct((NC, N_PAD, D_STRIDE), jnp.float32),
    )(xp, w1s)


def _relu_mm(p, b1e, w2e):
    def body(p_ref, b_ref, w_ref, o_ref):
        h = jnp.concatenate([p_ref[0], p_ref[1]], axis=-1)
        h = jnp.maximum(h + b_ref[...], 0.0)
        o_ref[...] = jnp.dot(h, w_ref[...], preferred_element_type=jnp.float32)
    return pl.pallas_call(
        body,
        grid=(8,),
        in_specs=[pl.BlockSpec((2, 1280, D_STRIDE), lambda i: (0, i, 0)),
                  pl.BlockSpec((1, 2 * D_STRIDE), lambda i: (0, 0)),
                  pl.BlockSpec((2 * D_STRIDE, C_STRIDE), lambda i: (0, 0))],
        out_specs=pl.BlockSpec((1280, C_STRIDE), lambda i: (i, 0)),
        out_shape=jax.ShapeDtypeStruct((N_PAD, C_STRIDE), jnp.float32),
    )(p, b1e.reshape(1, 2 * D_STRIDE), w2e)


def _lsm(q, b2p):
    def body(q_ref, b_ref, o_ref):
        z = q_ref[0] + q_ref[1] + b_ref[...]
        m = jnp.max(z, axis=1, keepdims=True)
        e = jnp.exp(z - m)
        o_ref[...] = z - m - jnp.log(jnp.sum(e, axis=1, keepdims=True))
    return pl.pallas_call(
        body,
        grid=(8,),
        in_specs=[pl.BlockSpec((2, 1280, C_STRIDE), lambda i: (0, i, 0)),
                  pl.BlockSpec((1, C_STRIDE), lambda i: (0, 0))],
        out_specs=pl.BlockSpec((1280, C_STRIDE), lambda i: (i, 0)),
        out_shape=jax.ShapeDtypeStruct((N_PAD, C_STRIDE), jnp.float32),
    )(q, b2p.reshape(1, C_STRIDE))


def kernel(x, edge_index, edge_weight, W1, b1, W2, b2):
    src = edge_index[0].astype(jnp.int32)
    dst = edge_index[1].astype(jnp.int32)
    w = edge_weight.astype(jnp.float32)

    src1 = jnp.pad(src, (0, E1_PAD - N_EDGES)).reshape(NS, NCH1, K)
    dst1 = jnp.pad(dst, (0, E1_PAD - N_EDGES)).reshape(NS, NCH1, K)
    w1e = jnp.pad(w, (0, E1_PAD - N_EDGES)).reshape(NS, NCH1, K)
    src2 = jnp.pad(src, (0, E2_PAD - N_EDGES)).reshape(NW, NCH2, K)
    dst2 = jnp.pad(dst, (0, E2_PAD - N_EDGES)).reshape(NW, NCH2, K)
    w2e = jnp.pad(w, (0, E2_PAD - N_EDGES)).reshape(NW, NCH2, K)

    xp = jnp.pad(x, ((0, N_PAD - N_NODES), (0, 0)))
    w1s = W1.reshape(D_FEAT, NC, D_HALF).transpose(1, 0, 2)
    w1s = jnp.pad(w1s, ((0, 0), (0, 0), (0, D_STRIDE - D_HALF)))
    t1 = _mm1(xp, w1s)
    p = _agg1(t1, src1, dst1, w1e)
    w2c = jnp.pad(W2, ((0, 0), (0, C_STRIDE - N_CLASSES)))
    w2x = jnp.zeros((2 * D_STRIDE, C_STRIDE), jnp.float32)
    w2x = w2x.at[0:D_HALF].set(w2c[0:D_HALF])
    w2x = w2x.at[D_STRIDE:D_STRIDE + D_HALF].set(w2c[D_HALF:D_FEAT])
    b1e = jnp.zeros((2 * D_STRIDE,), jnp.float32)
    b1e = b1e.at[0:D_HALF].set(b1[0:D_HALF])
    b1e = b1e.at[D_STRIDE:D_STRIDE + D_HALF].set(b1[D_HALF:D_FEAT])
    t2 = _relu_mm(p, b1e, w2x)
    q = _agg2(t2, src2, dst2, w2e)
    b2p = jnp.pad(b2, (0, C_STRIDE - N_CLASSES), constant_values=-1e30)
    out = _lsm(q, b2p)
    return out[:N_NODES, :N_CLASSES]
